# Initial kernel scaffold; baseline (speedup 1.0000x reference)
import jax
import jax.numpy as jnp
from jax.experimental import pallas as pl


def kernel(x, pe_table):
    raise NotImplementedError("write your pallas kernel here")


# speedup vs baseline: 2.1332x; 2.1332x over previous
import jax
import jax.numpy as jnp
from jax.experimental import pallas as pl

SEQ_BLK = 256


def _add_body(x_ref, pe_ref, out_ref):
    out_ref[...] = x_ref[...] + pe_ref[...][None, :, :]


def kernel(x, pe_table):
    batch, seq_len, d_model = x.shape
    pe = pe_table[:seq_len]
    grid = (seq_len // SEQ_BLK,)
    return pl.pallas_call(
        _add_body,
        grid=grid,
        in_specs=[
            pl.BlockSpec((batch, SEQ_BLK, d_model), lambda s: (0, s, 0)),
            pl.BlockSpec((SEQ_BLK, d_model), lambda s: (s, 0)),
        ],
        out_specs=pl.BlockSpec((batch, SEQ_BLK, d_model), lambda s: (0, s, 0)),
        out_shape=jax.ShapeDtypeStruct((batch, seq_len, d_model), x.dtype),
    )(x, pe)


# speedup vs baseline: 2.1630x; 1.0140x over previous
import jax
import jax.numpy as jnp
from jax.experimental import pallas as pl

SEQ_BLK = 512


def _add_body(x_ref, pe_ref, out_ref):
    out_ref[...] = x_ref[...] + pe_ref[...][None, :, :]


def kernel(x, pe_table):
    batch, seq_len, d_model = x.shape
    pe = pe_table[:seq_len]
    grid = (seq_len // SEQ_BLK,)
    return pl.pallas_call(
        _add_body,
        grid=grid,
        in_specs=[
            pl.BlockSpec((batch, SEQ_BLK, d_model), lambda s: (0, s, 0)),
            pl.BlockSpec((SEQ_BLK, d_model), lambda s: (s, 0)),
        ],
        out_specs=pl.BlockSpec((batch, SEQ_BLK, d_model), lambda s: (0, s, 0)),
        out_shape=jax.ShapeDtypeStruct((batch, seq_len, d_model), x.dtype),
    )(x, pe)
